# baseline (speedup 1.0000x reference)
import functools

import jax
import jax.numpy as jnp
from jax import lax
from jax.experimental import pallas as pl
from jax.experimental.pallas import tpu as pltpu
from jax.experimental.pallas import tpu_sc as plsc

N = 10000
D = 128
E = 320000
G = 16

NC = 2
NS = 16
NW = NC * NS

CHUNK = 128
NCHUNK = 80
GSZ = 8
GROUPS = NCHUNK // GSZ
EPAD = NW * NCHUNK * CHUNK
DEG_K = 8

NPAD = 10240
RPT = NPAD // NS
DPT = NPAD // NS

BLKA = 1024
NBLKA = NPAD // BLKA
BLK = 1000
NBLK = N // BLK


@functools.cache
def _sc_kernels():
    mesh = plsc.VectorSubcoreMesh(
        core_axis_name="c", subcore_axis_name="s",
        num_cores=NC, num_subcores=NS)

    sc_deg = pl.kernel(
        _sc_deg_body,
        out_type=jax.ShapeDtypeStruct((NC, NPAD), jnp.float32),
        mesh=mesh,
        scratch_types=[
            pltpu.VMEM((NCHUNK, CHUNK), jnp.int32),
            pltpu.VMEM((CHUNK,), jnp.float32),
            pltpu.VMEM((DPT,), jnp.float32),
            pltpu.VMEM_SHARED((NPAD,), jnp.float32),
            pltpu.SemaphoreType.DMA,
        ],
    )
    sc_agg = pl.kernel(
        _sc_agg_body,
        out_type=jax.ShapeDtypeStruct((NC, NPAD, D), jnp.float32),
        mesh=mesh,
        scratch_types=[
            pltpu.VMEM((3, GSZ, CHUNK), jnp.int32),
            pltpu.VMEM((3, GSZ, CHUNK), jnp.int32),
            pltpu.VMEM((CHUNK, D), jnp.float32),
            pltpu.VMEM((CHUNK, D), jnp.float32),
            pltpu.VMEM_SHARED((NPAD, D), jnp.float32),
            pltpu.SemaphoreType.DMA,
            pltpu.SemaphoreType.DMA,
            pltpu.SemaphoreType.DMA,
        ],
    )
    return sc_deg, sc_agg


def _sc_deg_body(dst_hbm, out_hbm, didx2, ones, zbuf, deg, sem):
    cid = lax.axis_index("c")
    sid = lax.axis_index("s")
    wid = cid * NS + sid

    pltpu.sync_copy(dst_hbm.at[wid], didx2)

    def fill_ones(i, _):
        ones[pl.ds(i * 16, 16)] = jnp.ones((16,), jnp.float32)
        return 0
    lax.fori_loop(0, CHUNK // 16, fill_ones, 0)

    def fill_zeros(i, _):
        zbuf[pl.ds(i * 16, 16)] = jnp.zeros((16,), jnp.float32)
        return 0
    lax.fori_loop(0, DPT // 16, fill_zeros, 0)

    pltpu.sync_copy(zbuf, deg.at[pl.ds(sid * DPT, DPT)])
    plsc.subcore_barrier()

    def body(j, _):
        descs = [
            pltpu.async_copy(ones, deg.at[didx2.at[j * DEG_K + t]], sem,
                             add=True)
            for t in range(DEG_K)
        ]
        for dsc in descs:
            dsc.wait()
        return 0
    lax.fori_loop(0, NCHUNK // DEG_K, body, 0)

    plsc.subcore_barrier()
    pltpu.sync_copy(deg.at[pl.ds(sid * DPT, DPT)],
                    out_hbm.at[cid, pl.ds(sid * DPT, DPT)])


def _sc_agg_body(u_hbm, src_hbm, dst_hbm, zer_hbm, out_hbm, sibuf, dibuf,
                 rows0, rows1, acc, sem, isem, ssem):
    cid = lax.axis_index("c")
    sid = lax.axis_index("s")
    wid = cid * NS + sid

    pltpu.sync_copy(src_hbm.at[wid, pl.ds(0, GSZ)], sibuf.at[0])
    pltpu.sync_copy(dst_hbm.at[wid, pl.ds(0, GSZ)], dibuf.at[0])
    for g0 in (1, 2):
        pltpu.async_copy(src_hbm.at[wid, pl.ds(g0 * GSZ, GSZ)],
                         sibuf.at[g0], isem)
        pltpu.async_copy(dst_hbm.at[wid, pl.ds(g0 * GSZ, GSZ)],
                         dibuf.at[g0], isem)

    zdescs = [
        pltpu.async_copy(zer_hbm, acc.at[pl.ds(sid * RPT + t * CHUNK, CHUNK)],
                         sem)
        for t in range(RPT // CHUNK)
    ]
    for dsc in zdescs:
        dsc.wait()
    pltpu.async_copy(u_hbm.at[sibuf.at[0, 0]], rows0, sem)
    plsc.subcore_barrier()

    def body(g, _):
        p = g % 3

        @pl.when(g < GROUPS - 1)
        def _wait_next_idx_group():
            pltpu.make_async_copy(
                src_hbm.at[wid, pl.ds((g + 1) * GSZ, GSZ)],
                sibuf.at[(g + 1) % 3], isem).wait()
            pltpu.make_async_copy(
                dst_hbm.at[wid, pl.ds((g + 1) * GSZ, GSZ)],
                dibuf.at[(g + 1) % 3], isem).wait()

        for c in range(GSZ):
            buf, obuf = (rows0, rows1) if c % 2 == 0 else (rows1, rows0)
            pltpu.make_async_copy(u_hbm.at[sibuf.at[p, c]], buf, sem).wait()
            pltpu.async_copy(buf, acc.at[dibuf.at[p, c]], ssem, add=True)
            if c > 0:
                pltpu.make_async_copy(obuf, acc.at[dibuf.at[p, c]],
                                      ssem).wait()
            else:
                @pl.when(g > 0)
                def _drain_prev_scatter():
                    pltpu.make_async_copy(obuf, acc.at[dibuf.at[p, c]],
                                          ssem).wait()
            if c < GSZ - 1:
                pltpu.async_copy(u_hbm.at[sibuf.at[p, c + 1]], obuf, sem)
            else:
                @pl.when(g < GROUPS - 1)
                def _fire_next_group():
                    pltpu.async_copy(u_hbm.at[sibuf.at[(g + 1) % 3, 0]],
                                     obuf, sem)

        @pl.when(g < GROUPS - 3)
        def _prefetch_indices():
            pltpu.async_copy(src_hbm.at[wid, pl.ds((g + 3) * GSZ, GSZ)],
                             sibuf.at[p], isem)
            pltpu.async_copy(dst_hbm.at[wid, pl.ds((g + 3) * GSZ, GSZ)],
                             dibuf.at[p], isem)
        return 0
    lax.fori_loop(0, GROUPS, body, 0)
    pltpu.make_async_copy(rows1, acc.at[dibuf.at[(GROUPS - 1) % 3, GSZ - 1]],
                          ssem).wait()

    plsc.subcore_barrier()
    pltpu.sync_copy(acc.at[pl.ds(sid * RPT, RPT)],
                    out_hbm.at[cid, pl.ds(sid * RPT, RPT)])


def _tc_a_body(x_ref, w1_ref, deg_ref, u1_ref, dinv_ref):
    dsum = deg_ref[0] + deg_ref[1] + 1.0
    dinv = lax.rsqrt(dsum)
    h = jnp.dot(x_ref[...], w1_ref[...], preferred_element_type=jnp.float32)
    u1_ref[...] = h * dinv
    dinv_ref[...] = dinv


def _tc_b_body(p_ref, u1_ref, dinv_ref, b1_ref, w2_ref, u2_ref):
    dinv = dinv_ref[...]
    s = p_ref[0] + p_ref[1] + u1_ref[...]
    z = jnp.maximum(s * dinv + b1_ref[...], 0.0)
    u2_ref[...] = jnp.dot(z, w2_ref[...], preferred_element_type=jnp.float32) * dinv


def _tc_c_body(p_ref, u2_ref, dinv_ref, b2_ref, batch_ref, out_ref, cnt_ref):
    i = pl.program_id(0)
    nb = pl.num_programs(0)
    z = (p_ref[0] + p_ref[1] + u2_ref[...]) * dinv_ref[...] + b2_ref[...]
    bt = batch_ref[0]
    oh = (lax.broadcasted_iota(jnp.int32, (G, BLK), 0) == bt).astype(jnp.float32)

    @pl.when(i == 0)
    def _init():
        out_ref[...] = jnp.zeros_like(out_ref)
        cnt_ref[...] = jnp.zeros_like(cnt_ref)

    out_ref[...] += jnp.dot(oh, z, preferred_element_type=jnp.float32)
    cnt_ref[...] += jnp.sum(oh, axis=1, keepdims=True)

    @pl.when(i == nb - 1)
    def _fin():
        out_ref[...] = out_ref[...] / jnp.maximum(cnt_ref[...], 1.0)


_tc_a = pl.pallas_call(
    _tc_a_body,
    grid=(NBLK,),
    in_specs=[
        pl.BlockSpec((BLK, D), lambda i: (i, 0)),
        pl.BlockSpec((D, D), lambda i: (0, 0)),
        pl.BlockSpec((NC, BLK, 1), lambda i: (0, i, 0)),
    ],
    out_specs=[
        pl.BlockSpec((BLK, D), lambda i: (i, 0)),
        pl.BlockSpec((BLK, 1), lambda i: (i, 0)),
    ],
    out_shape=[
        jax.ShapeDtypeStruct((NPAD, D), jnp.float32),
        jax.ShapeDtypeStruct((NPAD, 1), jnp.float32),
    ],
)

_tc_b = pl.pallas_call(
    _tc_b_body,
    grid=(NBLK,),
    in_specs=[
        pl.BlockSpec((NC, BLK, D), lambda i: (0, i, 0)),
        pl.BlockSpec((BLK, D), lambda i: (i, 0)),
        pl.BlockSpec((BLK, 1), lambda i: (i, 0)),
        pl.BlockSpec((1, D), lambda i: (0, 0)),
        pl.BlockSpec((D, D), lambda i: (0, 0)),
    ],
    out_specs=pl.BlockSpec((BLK, D), lambda i: (i, 0)),
    out_shape=jax.ShapeDtypeStruct((NPAD, D), jnp.float32),
)

_tc_c = pl.pallas_call(
    _tc_c_body,
    grid=(NBLK,),
    in_specs=[
        pl.BlockSpec((NC, BLK, D), lambda i: (0, i, 0)),
        pl.BlockSpec((BLK, D), lambda i: (i, 0)),
        pl.BlockSpec((BLK, 1), lambda i: (i, 0)),
        pl.BlockSpec((1, D), lambda i: (0, 0)),
        pl.BlockSpec((1, 1, BLK), lambda i: (i, 0, 0)),
    ],
    out_specs=pl.BlockSpec((G, D), lambda i: (0, 0)),
    out_shape=jax.ShapeDtypeStruct((G, D), jnp.float32),
    scratch_shapes=[pltpu.VMEM((G, 1), jnp.float32)],
)


def kernel(x, edge_index, batch, W1, b1, W2, b2):
    fill = N + jnp.arange(EPAD - E, dtype=jnp.int32) % (NPAD - N)
    src = jnp.concatenate([edge_index[0], fill]).reshape(NW, NCHUNK, CHUNK)
    dst = jnp.concatenate([edge_index[1], fill]).reshape(NW, NCHUNK, CHUNK)
    zer = jnp.zeros((CHUNK, D), jnp.float32)
    sc_deg, sc_agg = _sc_kernels()

    deg_parts = sc_deg(dst)
    degc = deg_parts.reshape(NC, NPAD, 1)

    u1, dinv = _tc_a(x, W1, degc)
    parts1 = sc_agg(u1, src, dst, zer)
    u2 = _tc_b(parts1, u1, dinv, b1.reshape(1, D), W2)
    parts2 = sc_agg(u2, src, dst, zer)
    out = _tc_c(parts2, u2, dinv, b2.reshape(1, D),
                batch.reshape(NBLK, 1, BLK))
    return out


# speedup vs baseline: 1.1489x; 1.1489x over previous
import functools

import jax
import jax.numpy as jnp
from jax import lax
from jax.experimental import pallas as pl
from jax.experimental.pallas import tpu as pltpu
from jax.experimental.pallas import tpu_sc as plsc

N = 10000
D = 128
E = 320000
G = 16

NC = 2
NS = 16
NW = NC * NS

CHUNK = 128
NCHUNK = 80
GSZ = 8
GROUPS = NCHUNK // GSZ
EPAD = NW * NCHUNK * CHUNK
DEG_K = 8

NPAD = 10240
RPT = NPAD // NS
DPT = NPAD // NS

BLKA = 1024
NBLKA = NPAD // BLKA
BLK = 1000
NBLK = N // BLK


@functools.cache
def _sc_kernels():
    mesh = plsc.VectorSubcoreMesh(
        core_axis_name="c", subcore_axis_name="s",
        num_cores=NC, num_subcores=NS)

    sc_deg = pl.kernel(
        _sc_deg_body,
        out_type=jax.ShapeDtypeStruct((NC, NPAD), jnp.float32),
        mesh=mesh,
        scratch_types=[
            pltpu.VMEM((NCHUNK, CHUNK), jnp.int32),
            pltpu.VMEM((CHUNK,), jnp.float32),
            pltpu.VMEM((DPT,), jnp.float32),
            pltpu.VMEM_SHARED((NPAD,), jnp.float32),
            pltpu.SemaphoreType.DMA,
        ],
    )
    sc_agg = pl.kernel(
        _sc_agg_body,
        out_type=jax.ShapeDtypeStruct((NC, NPAD, D), jnp.float32),
        mesh=mesh,
        scratch_types=[
            pltpu.VMEM((3, GSZ, CHUNK), jnp.int32),
            pltpu.VMEM((3, GSZ, CHUNK), jnp.int32),
            pltpu.VMEM((CHUNK, D), jnp.float32),
            pltpu.VMEM((CHUNK, D), jnp.float32),
            pltpu.VMEM_SHARED((NPAD, D), jnp.float32),
            pltpu.SemaphoreType.DMA,
            pltpu.SemaphoreType.DMA,
        ],
    )
    return sc_deg, sc_agg


def _sc_deg_body(dst_hbm, out_hbm, didx2, ones, zbuf, deg, sem):
    cid = lax.axis_index("c")
    sid = lax.axis_index("s")
    wid = cid * NS + sid

    pltpu.sync_copy(dst_hbm.at[wid], didx2)

    def fill_ones(i, _):
        ones[pl.ds(i * 16, 16)] = jnp.ones((16,), jnp.float32)
        return 0
    lax.fori_loop(0, CHUNK // 16, fill_ones, 0)

    def fill_zeros(i, _):
        zbuf[pl.ds(i * 16, 16)] = jnp.zeros((16,), jnp.float32)
        return 0
    lax.fori_loop(0, DPT // 16, fill_zeros, 0)

    pltpu.sync_copy(zbuf, deg.at[pl.ds(sid * DPT, DPT)])
    plsc.subcore_barrier()

    def body(j, _):
        descs = [
            pltpu.async_copy(ones, deg.at[didx2.at[j * DEG_K + t]], sem,
                             add=True)
            for t in range(DEG_K)
        ]
        for dsc in descs:
            dsc.wait()
        return 0
    lax.fori_loop(0, NCHUNK // DEG_K, body, 0)

    plsc.subcore_barrier()
    pltpu.sync_copy(deg.at[pl.ds(sid * DPT, DPT)],
                    out_hbm.at[cid, pl.ds(sid * DPT, DPT)])


def _sc_agg_body(u_hbm, src_hbm, dst_hbm, zer_hbm, out_hbm, sibuf, dibuf,
                 rows0, rows1, acc, sem, isem):
    cid = lax.axis_index("c")
    sid = lax.axis_index("s")
    wid = cid * NS + sid

    pltpu.sync_copy(src_hbm.at[wid, pl.ds(0, GSZ)], sibuf.at[0])
    pltpu.sync_copy(dst_hbm.at[wid, pl.ds(0, GSZ)], dibuf.at[0])
    for g0 in (1, 2):
        pltpu.async_copy(src_hbm.at[wid, pl.ds(g0 * GSZ, GSZ)],
                         sibuf.at[g0], isem)
        pltpu.async_copy(dst_hbm.at[wid, pl.ds(g0 * GSZ, GSZ)],
                         dibuf.at[g0], isem)

    zdescs = [
        pltpu.async_copy(zer_hbm, acc.at[pl.ds(sid * RPT + t * CHUNK, CHUNK)],
                         sem)
        for t in range(RPT // CHUNK)
    ]
    for dsc in zdescs:
        dsc.wait()
    pltpu.async_copy(u_hbm.at[sibuf.at[0, 0]], rows0, sem)
    plsc.subcore_barrier()

    def body(g, _):
        p = g % 3

        @pl.when(g < GROUPS - 1)
        def _wait_next_idx_group():
            pltpu.make_async_copy(
                src_hbm.at[wid, pl.ds((g + 1) * GSZ, GSZ)],
                sibuf.at[(g + 1) % 3], isem).wait()
            pltpu.make_async_copy(
                dst_hbm.at[wid, pl.ds((g + 1) * GSZ, GSZ)],
                dibuf.at[(g + 1) % 3], isem).wait()

        for c in range(GSZ):
            buf, nbuf = (rows0, rows1) if c % 2 == 0 else (rows1, rows0)
            if c < GSZ - 1:
                pltpu.async_copy(u_hbm.at[sibuf.at[p, c + 1]], nbuf, sem)
            else:
                @pl.when(g < GROUPS - 1)
                def _fire_next_group():
                    pltpu.async_copy(u_hbm.at[sibuf.at[(g + 1) % 3, 0]],
                                     nbuf, sem)
            pltpu.make_async_copy(u_hbm.at[sibuf.at[p, c]], buf, sem).wait()
            pltpu.sync_copy(buf, acc.at[dibuf.at[p, c]], add=True)

        @pl.when(g < GROUPS - 3)
        def _prefetch_indices():
            pltpu.async_copy(src_hbm.at[wid, pl.ds((g + 3) * GSZ, GSZ)],
                             sibuf.at[p], isem)
            pltpu.async_copy(dst_hbm.at[wid, pl.ds((g + 3) * GSZ, GSZ)],
                             dibuf.at[p], isem)
        return 0
    lax.fori_loop(0, GROUPS, body, 0)

    plsc.subcore_barrier()
    pltpu.sync_copy(acc.at[pl.ds(sid * RPT, RPT)],
                    out_hbm.at[cid, pl.ds(sid * RPT, RPT)])


def _tc_a_body(x_ref, w1_ref, deg_ref, u1_ref, dinv_ref):
    dsum = deg_ref[0] + deg_ref[1] + 1.0
    dinv = lax.rsqrt(dsum)
    h = jnp.dot(x_ref[...], w1_ref[...], preferred_element_type=jnp.float32)
    u1_ref[...] = h * dinv
    dinv_ref[...] = dinv


def _tc_b_body(p_ref, u1_ref, dinv_ref, b1_ref, w2_ref, u2_ref):
    dinv = dinv_ref[...]
    s = p_ref[0] + p_ref[1] + u1_ref[...]
    z = jnp.maximum(s * dinv + b1_ref[...], 0.0)
    u2_ref[...] = jnp.dot(z, w2_ref[...], preferred_element_type=jnp.float32) * dinv


def _tc_c_body(p_ref, u2_ref, dinv_ref, b2_ref, batch_ref, out_ref, cnt_ref):
    i = pl.program_id(0)
    nb = pl.num_programs(0)
    z = (p_ref[0] + p_ref[1] + u2_ref[...]) * dinv_ref[...] + b2_ref[...]
    bt = batch_ref[0]
    oh = (lax.broadcasted_iota(jnp.int32, (G, BLK), 0) == bt).astype(jnp.float32)

    @pl.when(i == 0)
    def _init():
        out_ref[...] = jnp.zeros_like(out_ref)
        cnt_ref[...] = jnp.zeros_like(cnt_ref)

    out_ref[...] += jnp.dot(oh, z, preferred_element_type=jnp.float32)
    cnt_ref[...] += jnp.sum(oh, axis=1, keepdims=True)

    @pl.when(i == nb - 1)
    def _fin():
        out_ref[...] = out_ref[...] / jnp.maximum(cnt_ref[...], 1.0)


_tc_a = pl.pallas_call(
    _tc_a_body,
    grid=(NBLK,),
    in_specs=[
        pl.BlockSpec((BLK, D), lambda i: (i, 0)),
        pl.BlockSpec((D, D), lambda i: (0, 0)),
        pl.BlockSpec((NC, BLK, 1), lambda i: (0, i, 0)),
    ],
    out_specs=[
        pl.BlockSpec((BLK, D), lambda i: (i, 0)),
        pl.BlockSpec((BLK, 1), lambda i: (i, 0)),
    ],
    out_shape=[
        jax.ShapeDtypeStruct((NPAD, D), jnp.float32),
        jax.ShapeDtypeStruct((NPAD, 1), jnp.float32),
    ],
)

_tc_b = pl.pallas_call(
    _tc_b_body,
    grid=(NBLK,),
    in_specs=[
        pl.BlockSpec((NC, BLK, D), lambda i: (0, i, 0)),
        pl.BlockSpec((BLK, D), lambda i: (i, 0)),
        pl.BlockSpec((BLK, 1), lambda i: (i, 0)),
        pl.BlockSpec((1, D), lambda i: (0, 0)),
        pl.BlockSpec((D, D), lambda i: (0, 0)),
    ],
    out_specs=pl.BlockSpec((BLK, D), lambda i: (i, 0)),
    out_shape=jax.ShapeDtypeStruct((NPAD, D), jnp.float32),
)

_tc_c = pl.pallas_call(
    _tc_c_body,
    grid=(NBLK,),
    in_specs=[
        pl.BlockSpec((NC, BLK, D), lambda i: (0, i, 0)),
        pl.BlockSpec((BLK, D), lambda i: (i, 0)),
        pl.BlockSpec((BLK, 1), lambda i: (i, 0)),
        pl.BlockSpec((1, D), lambda i: (0, 0)),
        pl.BlockSpec((1, 1, BLK), lambda i: (i, 0, 0)),
    ],
    out_specs=pl.BlockSpec((G, D), lambda i: (0, 0)),
    out_shape=jax.ShapeDtypeStruct((G, D), jnp.float32),
    scratch_shapes=[pltpu.VMEM((G, 1), jnp.float32)],
)


def kernel(x, edge_index, batch, W1, b1, W2, b2):
    fill = N + jnp.arange(EPAD - E, dtype=jnp.int32) % (NPAD - N)
    src = jnp.concatenate([edge_index[0], fill]).reshape(NW, NCHUNK, CHUNK)
    dst = jnp.concatenate([edge_index[1], fill]).reshape(NW, NCHUNK, CHUNK)
    zer = jnp.zeros((CHUNK, D), jnp.float32)
    sc_deg, sc_agg = _sc_kernels()

    deg_parts = sc_deg(dst)
    degc = deg_parts.reshape(NC, NPAD, 1)

    u1, dinv = _tc_a(x, W1, degc)
    parts1 = sc_agg(u1, src, dst, zer)
    u2 = _tc_b(parts1, u1, dinv, b1.reshape(1, D), W2)
    parts2 = sc_agg(u2, src, dst, zer)
    out = _tc_c(parts2, u2, dinv, b2.reshape(1, D),
                batch.reshape(NBLK, 1, BLK))
    return out
